# Initial kernel scaffold; baseline (speedup 1.0000x reference)
import jax
import jax.numpy as jnp
from jax.experimental import pallas as pl


def kernel(x, weight):
    raise NotImplementedError("write your pallas kernel here")


# speedup vs baseline: 26.9403x; 26.9403x over previous
import functools

import jax
import jax.numpy as jnp
from jax import lax
from jax.experimental import pallas as pl
from jax.experimental.pallas import tpu as pltpu
from jax.experimental.pallas import tpu_sc as plsc

F = 100
E = 64
NE = 100
ROWS = F * NE
L = 16
NC = 2
NS = 16
NW = NC * NS

C = 8
CF = C * F
NVE = CF // L
G = 80
NG = CF // G
RU = 5


def _scale_table_body(w_ref, o_ref):
    w = w_ref[...]
    norm = jnp.sqrt(jnp.sum(w * w, axis=1, keepdims=True))
    scale = jnp.where(norm > 1.0, 1.0 / (norm + 1e-7), 1.0) * (1.0 / F)
    o_ref[...] = w * scale


def _scale_table(weight):
    return pl.pallas_call(
        _scale_table_body,
        grid=(10,),
        in_specs=[pl.BlockSpec((ROWS // 10, E), lambda i: (i, 0))],
        out_specs=pl.BlockSpec((ROWS // 10, E), lambda i: (i, 0)),
        out_shape=jax.ShapeDtypeStruct((ROWS, E), jnp.float32),
    )(weight)


def _make_sc_kernel(n_tok):
    tpw = n_tok // NW
    chunks = tpw // C
    mesh = plsc.VectorSubcoreMesh(core_axis_name="c", subcore_axis_name="s")

    @functools.partial(
        pl.kernel,
        mesh=mesh,
        compiler_params=pltpu.CompilerParams(use_tc_tiling_on_sc=False),
        out_type=jax.ShapeDtypeStruct((n_tok, E), jnp.float32),
        scratch_types=[
            pltpu.VMEM((CF,), jnp.float32),
            pltpu.VMEM((CF,), jnp.int32),
            pltpu.VMEM((CF,), jnp.int32),
            pltpu.VMEM((CF, E), jnp.float32),
            pltpu.VMEM((C, E), jnp.float32),
            pltpu.SemaphoreType.DMA,
        ],
    )
    def sc_gather(x_hbm, table_hbm, out_hbm, x_v, idx_v, off_v, rows_v,
                  out_v, sem):
        wid = lax.axis_index("c") * NS + lax.axis_index("s")
        base_tok = wid * tpw

        def _off_body(v, carry):
            j = lax.iota(jnp.int32, L) + v * L
            off_v[pl.ds(v * L, L)] = (j % F) * NE
            return carry

        lax.fori_loop(0, NVE, _off_body, 0)

        def _chunk(g, carry):
            tok0 = base_tok + g * C
            pltpu.sync_copy(x_hbm.at[pl.ds(tok0 * F, CF)], x_v)

            def _idx_body(v, c2):
                xe = x_v[pl.ds(v * L, L)]
                xi = jnp.clip((xe + 50.0).astype(jnp.int32), 0, NE - 1)
                idx_v[pl.ds(v * L, L)] = xi + off_v[pl.ds(v * L, L)]
                return c2

            lax.fori_loop(0, NVE, _idx_body, 0)

            cps = [
                pltpu.make_async_copy(
                    table_hbm.at[idx_v.at[pl.ds(i * G, G)]],
                    rows_v.at[pl.ds(i * G, G)],
                    sem,
                )
                for i in range(NG)
            ]
            for cp in cps:
                cp.start()
            for cp in cps:
                cp.wait()

            for t in range(C):
                def _red(q, acc):
                    r0 = t * F + q * RU
                    for u in range(RU):
                        acc = tuple(
                            acc[c] + rows_v[r0 + u, pl.ds(c * L, L)]
                            for c in range(E // L)
                        )
                    return acc

                z = jnp.zeros((L,), jnp.float32)
                acc = lax.fori_loop(0, F // RU, _red, (z, z, z, z))
                for c in range(E // L):
                    out_v[t, pl.ds(c * L, L)] = acc[c]

            pltpu.sync_copy(out_v, out_hbm.at[pl.ds(tok0, C)])
            return carry

        lax.fori_loop(0, chunks, _chunk, 0)

    return sc_gather


def kernel(x, weight):
    b, s, f = x.shape
    n_tok = b * s
    scaled = _scale_table(weight)
    out = _make_sc_kernel(n_tok)(x.reshape(n_tok * f), scaled)
    return out.reshape(b, s, E)


# speedup vs baseline: 108.3905x; 4.0234x over previous
import functools

import jax
import jax.numpy as jnp
from jax import lax
from jax.experimental import pallas as pl
from jax.experimental.pallas import tpu as pltpu
from jax.experimental.pallas import tpu_sc as plsc

F = 100
E = 64
NE = 100
ROWS = F * NE
L = 16
NC = 2
NS = 16
NW = NC * NS

C = 8
CF = C * F
NVE = CF // L
G = 80
NG = CF // G
RU = 4


def _scale_table_body(w_ref, o_ref):
    w = w_ref[...]
    norm = jnp.sqrt(jnp.sum(w * w, axis=1, keepdims=True))
    scale = jnp.where(norm > 1.0, 1.0 / (norm + 1e-7), 1.0) * (1.0 / F)
    o_ref[...] = w * scale


def _scale_table(weight):
    return pl.pallas_call(
        _scale_table_body,
        grid=(10,),
        in_specs=[pl.BlockSpec((ROWS // 10, E), lambda i: (i, 0))],
        out_specs=pl.BlockSpec((ROWS // 10, E), lambda i: (i, 0)),
        out_shape=jax.ShapeDtypeStruct((ROWS, E), jnp.float32),
    )(weight)


def _make_sc_kernel(n_tok):
    tpw = n_tok // NW
    chunks = tpw // C
    mesh = plsc.VectorSubcoreMesh(core_axis_name="c", subcore_axis_name="s")

    @functools.partial(
        pl.kernel,
        mesh=mesh,
        compiler_params=pltpu.CompilerParams(use_tc_tiling_on_sc=False,
                                             needs_layout_passes=False),
        out_type=jax.ShapeDtypeStruct((n_tok, E), jnp.float32),
        scratch_types=[
            pltpu.VMEM((2, CF), jnp.float32),
            pltpu.VMEM((2, CF), jnp.int32),
            pltpu.VMEM((CF,), jnp.int32),
            pltpu.VMEM((2, CF, E), jnp.bfloat16),
            pltpu.VMEM((2, C, E), jnp.float32),
            pltpu.VMEM_SHARED((ROWS, E), jnp.bfloat16),
            pltpu.SemaphoreType.DMA,
            pltpu.SemaphoreType.DMA,
            pltpu.SemaphoreType.DMA,
            pltpu.SemaphoreType.DMA,
            pltpu.SemaphoreType.DMA,
            pltpu.SemaphoreType.DMA,
        ],
    )
    def sc_gather(x_hbm, table_hbm, out_hbm, x_v, idx_v, off_v, rows_v,
                  out_v, table_sh, semr0, semr1, semx0, semx1, semo0, semo1):
        wid = lax.axis_index("c") * NS + lax.axis_index("s")
        base_tok = wid * tpw
        semr = (semr0, semr1)
        semx = (semx0, semx1)
        semo = (semo0, semo1)

        @pl.when(lax.axis_index("s") == 0)
        def _stage():
            pltpu.sync_copy(table_hbm, table_sh)

        plsc.subcore_barrier()

        def _off_body(v, carry):
            j = lax.iota(jnp.int32, L) + v * L
            off_v[pl.ds(v * L, L)] = (j % F) * NE
            return carry

        lax.fori_loop(0, NVE, _off_body, 0)

        def _x_copy(g, b):
            return pltpu.make_async_copy(
                x_hbm.at[pl.ds((base_tok + g * C) * F, CF)],
                x_v.at[b], semx[b])

        def _out_copy(g, b):
            return pltpu.make_async_copy(
                out_v.at[b], out_hbm.at[pl.ds(base_tok + g * C, C)], semo[b])

        def _gathers(b):
            return [
                pltpu.make_async_copy(
                    table_sh.at[idx_v.at[b, pl.ds(i * G, G)]],
                    rows_v.at[b, pl.ds(i * G, G)],
                    semr[b],
                )
                for i in range(NG)
            ]

        def _prep(g, b):
            def _idx_body(v2, c2):
                for w in range(2):
                    v = v2 * 2 + w
                    xe = x_v[b, pl.ds(v * L, L)]
                    xi = jnp.clip((xe + 50.0).astype(jnp.int32), 0, NE - 1)
                    idx_v[b, pl.ds(v * L, L)] = xi + off_v[pl.ds(v * L, L)]
                return c2

            lax.fori_loop(0, NVE // 2, _idx_body, 0)
            for cp in _gathers(b):
                cp.start()

        def _pair(i, carry):
            for b in (0, 1):
                g = i * 2 + b
                nb = 1 - b

                @pl.when(g + 1 < chunks)
                def _():
                    _x_copy(g + 1, nb).wait()
                    _prep(g + 1, nb)

                @pl.when(g + 2 < chunks)
                def _():
                    _x_copy(g + 2, b).start()

                for cp in _gathers(b):
                    cp.wait()

                @pl.when(g >= 2)
                def _():
                    _out_copy(g - 2, b).wait()

                def _red(q, acc):
                    na = []
                    for t in range(C):
                        r0 = t * F + q * RU
                        vs = [acc[t * 4 + c] for c in range(4)]
                        for half in range(2):
                            sl = pl.ds(half * 2 * L, 2 * L)
                            h = ((rows_v[b, r0, sl] + rows_v[b, r0 + 1, sl])
                                 + (rows_v[b, r0 + 2, sl]
                                    + rows_v[b, r0 + 3, sl]))
                            a0, a1 = plsc.unpack(
                                h, format=plsc.PackFormat.INTERLEAVED)
                            vs[half * 2] = vs[half * 2] + a0
                            vs[half * 2 + 1] = vs[half * 2 + 1] + a1
                        na.extend(vs)
                    return tuple(na)

                z = jnp.zeros((L,), jnp.float32)
                acc = lax.fori_loop(0, F // RU, _red, (z,) * (4 * C))
                for t in range(C):
                    for c in range(E // L):
                        out_v[b, t, pl.ds(c * L, L)] = acc[t * 4 + c]

                _out_copy(g, b).start()
            return carry

        _x_copy(0, 0).start()
        _x_copy(0, 0).wait()
        _prep(0, 0)
        _x_copy(1, 1).start()
        lax.fori_loop(0, chunks // 2, _pair, 0)
        _out_copy(chunks - 2, 0).wait()
        _out_copy(chunks - 1, 1).wait()

    return sc_gather


def kernel(x, weight):
    b, s, f = x.shape
    n_tok = b * s
    scaled = _scale_table(weight)
    stored = (scaled.reshape(ROWS, 2, 2, L).transpose(0, 1, 3, 2)
              .reshape(ROWS, E).astype(jnp.bfloat16))
    out = _make_sc_kernel(n_tok)(x.reshape(n_tok * f), stored)
    return out.reshape(b, s, E)


# speedup vs baseline: 113.4437x; 1.0466x over previous
import functools

import jax
import jax.numpy as jnp
from jax import lax
from jax.experimental import pallas as pl
from jax.experimental.pallas import tpu as pltpu
from jax.experimental.pallas import tpu_sc as plsc

F = 100
E = 64
NE = 100
ROWS = F * NE
L = 16
NC = 2
NS = 16
NW = NC * NS

C = 8
CF = C * F
NVE = CF // L
G = 80
NG = CF // G
RU = 4


def _scale_table_body(w_ref, o_ref):
    w = w_ref[...]
    norm = jnp.sqrt(jnp.sum(w * w, axis=1, keepdims=True))
    scale = jnp.where(norm > 1.0, 1.0 / (norm + 1e-7), 1.0) * (1.0 / F)
    o_ref[...] = w * scale


def _scale_table(weight):
    return pl.pallas_call(
        _scale_table_body,
        grid=(10,),
        in_specs=[pl.BlockSpec((ROWS // 10, E), lambda i: (i, 0))],
        out_specs=pl.BlockSpec((ROWS // 10, E), lambda i: (i, 0)),
        out_shape=jax.ShapeDtypeStruct((ROWS, E), jnp.float32),
    )(weight)


def _make_sc_kernel(n_tok):
    tpw = n_tok // NW
    chunks = tpw // C
    mesh = plsc.VectorSubcoreMesh(core_axis_name="c", subcore_axis_name="s")

    @functools.partial(
        pl.kernel,
        mesh=mesh,
        compiler_params=pltpu.CompilerParams(use_tc_tiling_on_sc=False,
                                             needs_layout_passes=False),
        out_type=jax.ShapeDtypeStruct((n_tok, E), jnp.float32),
        scratch_types=[
            pltpu.VMEM((2, CF), jnp.float32),
            pltpu.VMEM((2, CF), jnp.int32),
            pltpu.VMEM((CF,), jnp.int32),
            pltpu.VMEM((2, CF, E), jnp.bfloat16),
            pltpu.VMEM((2, C, E), jnp.float32),
            pltpu.VMEM_SHARED((ROWS, E), jnp.bfloat16),
            pltpu.SemaphoreType.DMA,
            pltpu.SemaphoreType.DMA,
            pltpu.SemaphoreType.DMA,
            pltpu.SemaphoreType.DMA,
            pltpu.SemaphoreType.DMA,
            pltpu.SemaphoreType.DMA,
        ],
    )
    def sc_gather(x_hbm, table_hbm, out_hbm, x_v, idx_v, off_v, rows_v,
                  out_v, table_sh, semr0, semr1, semx0, semx1, semo0, semo1):
        wid = lax.axis_index("c") * NS + lax.axis_index("s")
        base_tok = wid * tpw
        semr = (semr0, semr1)
        semx = (semx0, semx1)
        semo = (semo0, semo1)

        @pl.when(lax.axis_index("s") == 0)
        def _stage():
            pltpu.sync_copy(table_hbm, table_sh)

        plsc.subcore_barrier()

        def _off_body(v, carry):
            j = lax.iota(jnp.int32, L) + v * L
            off_v[pl.ds(v * L, L)] = (j % F) * NE
            return carry

        lax.fori_loop(0, NVE, _off_body, 0)

        def _x_copy(g, b):
            return pltpu.make_async_copy(
                x_hbm.at[pl.ds((base_tok + g * C) * F, CF)],
                x_v.at[b], semx[b])

        def _out_copy(g, b):
            return pltpu.make_async_copy(
                out_v.at[b], out_hbm.at[pl.ds(base_tok + g * C, C)], semo[b])

        def _gathers(b):
            return [
                pltpu.make_async_copy(
                    table_sh.at[idx_v.at[b, pl.ds(i * G, G)]],
                    rows_v.at[b, pl.ds(i * G, G)],
                    semr[b],
                )
                for i in range(NG)
            ]

        def _prep(g, b):
            def _idx_body(v2, c2):
                for w in range(2):
                    v = v2 * 2 + w
                    xe = x_v[b, pl.ds(v * L, L)]
                    xi = jnp.clip((xe + 50.0).astype(jnp.int32), 0, NE - 1)
                    idx_v[b, pl.ds(v * L, L)] = xi + off_v[pl.ds(v * L, L)]
                return c2

            lax.fori_loop(0, NVE // 2, _idx_body, 0)
            for cp in _gathers(b):
                cp.start()

        def _pair(i, carry):
            for b in (0, 1):
                g = i * 2 + b
                nb = 1 - b

                @pl.when(g + 1 < chunks)
                def _():
                    _x_copy(g + 1, nb).wait()
                    _prep(g + 1, nb)

                @pl.when(g + 2 < chunks)
                def _():
                    _x_copy(g + 2, b).start()

                for cp in _gathers(b):
                    cp.wait()

                @pl.when(g >= 2)
                def _():
                    _out_copy(g - 2, b).wait()

                for t in range(C):
                    def _red(q, acc):
                        r0 = t * F + q * RU
                        for u in range(0, RU, 2):
                            h0 = (rows_v[b, r0 + u, pl.ds(0, 2 * L)]
                                  + rows_v[b, r0 + u + 1, pl.ds(0, 2 * L)])
                            h1 = (rows_v[b, r0 + u, pl.ds(2 * L, 2 * L)]
                                  + rows_v[b, r0 + u + 1, pl.ds(2 * L, 2 * L)])
                            a0, a1 = plsc.unpack(
                                h0, format=plsc.PackFormat.INTERLEAVED)
                            a2, a3 = plsc.unpack(
                                h1, format=plsc.PackFormat.INTERLEAVED)
                            acc = (acc[0] + a0, acc[1] + a1,
                                   acc[2] + a2, acc[3] + a3)
                        return acc

                    z = jnp.zeros((L,), jnp.float32)
                    acc = lax.fori_loop(0, F // RU, _red, (z, z, z, z))
                    for c in range(E // L):
                        out_v[b, t, pl.ds(c * L, L)] = acc[c]

                _out_copy(g, b).start()
            return carry

        _x_copy(0, 0).start()
        _x_copy(0, 0).wait()
        _prep(0, 0)
        _x_copy(1, 1).start()
        lax.fori_loop(0, chunks // 2, _pair, 0)
        _out_copy(chunks - 2, 0).wait()
        _out_copy(chunks - 1, 1).wait()

    return sc_gather


def kernel(x, weight):
    b, s, f = x.shape
    n_tok = b * s
    scaled = _scale_table(weight)
    stored = (scaled.reshape(ROWS, 2, 2, L).transpose(0, 1, 3, 2)
              .reshape(ROWS, E).astype(jnp.bfloat16))
    out = _make_sc_kernel(n_tok)(x.reshape(n_tok * f), stored)
    return out.reshape(b, s, E)
